# baseline (speedup 1.0000x reference)
import jax
import jax.numpy as jnp
from jax import lax
from jax.experimental import pallas as pl
from jax.experimental.pallas import tpu as pltpu
from jax.experimental.pallas import tpu_sc as plsc

B, H, KV, DH = 16, 16, 4096, 128
BH = B * H
NC, NS = 2, 16
NW = NC * NS
RPW = (BH * KV) // NW
CH = 256
PAIRS = RPW // (2 * CH)
SC_BH = BH // NW


def _sc_full_body(cache_hbm, cur_hbm, rows_hbm, out_hbm,
                  buf0, buf1, idx_v, crow_v, in_sems, out_sems, ssem):
    wid = lax.axis_index("s") * NC + lax.axis_index("c")
    base = wid * RPW
    bufs = (buf0, buf1)

    @pl.loop(0, PAIRS)
    def _(cc):
        c0 = base + (2 * cc) * CH
        ins = [
            pltpu.make_async_copy(
                cache_hbm.at[pl.ds(c0 + b * CH, CH)], bufs[b], in_sems.at[b])
            for b in range(2)
        ]
        outs = [
            pltpu.make_async_copy(
                bufs[b], out_hbm.at[pl.ds(c0 + b * CH, CH)], out_sems.at[b])
            for b in range(2)
        ]
        for cp in ins:
            cp.start()
        for cp in ins:
            cp.wait()
        for cp in outs:
            cp.start()
        for cp in outs:
            cp.wait()

    sbase = wid * SC_BH
    pltpu.sync_copy(rows_hbm.at[pl.ds(sbase, SC_BH)], idx_v)
    pltpu.sync_copy(cur_hbm.at[pl.ds(sbase, SC_BH)], crow_v)
    pltpu.async_copy(crow_v, out_hbm.at[idx_v], ssem).wait()


_sc_full = pl.kernel(
    _sc_full_body,
    out_type=jax.ShapeDtypeStruct((BH * KV, DH), jnp.float32),
    mesh=plsc.VectorSubcoreMesh(core_axis_name="c", subcore_axis_name="s"),
    scratch_types=[
        pltpu.VMEM((CH, DH), jnp.float32),
        pltpu.VMEM((CH, DH), jnp.float32),
        pltpu.VMEM((SC_BH,), jnp.int32),
        pltpu.VMEM((SC_BH, DH), jnp.float32),
        pltpu.SemaphoreType.DMA((2,)),
        pltpu.SemaphoreType.DMA((2,)),
        pltpu.SemaphoreType.DMA,
    ],
)


def kernel(cur, dim, idx, cache):
    del dim
    cache2 = cache.reshape(BH * KV, DH)
    cur2 = cur.reshape(BH, DH)
    rows = jnp.arange(BH, dtype=jnp.int32) * KV + (idx[0] - 1)
    out = _sc_full(cache2, cur2, rows)
    return out.reshape(B, H, KV, DH)


# speedup vs baseline: 1.2024x; 1.2024x over previous
import jax
import jax.numpy as jnp
from jax.experimental import pallas as pl
from jax.experimental.pallas import tpu as pltpu

B, H, KV, DH = 16, 16, 4096, 128
BH = B * H


def _copy_patch_kernel(idx_ref, cur_ref, cache_ref, out_ref):
    out_ref[...] = cache_ref[...]
    bh_blk, kv_blk = out_ref.shape[0], out_ref.shape[1]
    i = pl.program_id(0)
    j = pl.program_id(1)
    slot = idx_ref[0] - 1
    off = slot - j * kv_blk

    @pl.when((off >= 0) & (off < kv_blk))
    def _():
        out_ref[:, pl.ds(off, 1), :] = cur_ref[pl.ds(i * bh_blk, bh_blk), :, :]


def kernel(cur, dim, idx, cache):
    del dim
    cache3 = cache.reshape(BH, KV, DH)
    cur3 = cur.reshape(BH, 1, DH)

    bh_blk = min(8, BH)
    kv_blk = min(2048, KV)
    grid = (BH // bh_blk, KV // kv_blk)

    out = pl.pallas_call(
        _copy_patch_kernel,
        grid=grid,
        in_specs=[
            pl.BlockSpec(memory_space=pltpu.SMEM),
            pl.BlockSpec((BH, 1, DH), lambda i, j: (0, 0, 0)),
            pl.BlockSpec((bh_blk, kv_blk, DH), lambda i, j: (i, j, 0)),
        ],
        out_specs=pl.BlockSpec((bh_blk, kv_blk, DH), lambda i, j: (i, j, 0)),
        out_shape=jax.ShapeDtypeStruct((BH, KV, DH), cache.dtype),
        compiler_params=pltpu.CompilerParams(
            dimension_semantics=("arbitrary", "arbitrary"),
            vmem_limit_bytes=63 * 1024 * 1024,
        ),
    )(idx, cur3, cache3)
    return out.reshape(B, H, KV, DH)
